# baseline (speedup 1.0000x reference)
import functools
import jax
import jax.numpy as jnp
from jax import lax
from jax.experimental import pallas as pl
from jax.experimental.pallas import tpu as pltpu, tpu_sc as plsc

N = 10000
E = 320000
G = 128
D = 128
NC, NS, L = 2, 16, 16
NW = NC * NS
CH = 128
NCHUNK = E // CH
NROW_CH = N // CH
NP = 10112
NROW_CHP = NP // CH
NCHP = 2504
NAGG = N + 16
DUMMY = N + 8
BLK = 400
NBLK = N // BLK
EPS = 1e-5
F32 = jnp.float32


def _wid():
    return lax.axis_index("s") * NC + lax.axis_index("c")


def _zero_vmem(buf, rows):
    z16 = jnp.zeros((L,), F32)

    @pl.loop(0, rows)
    def _z(i):
        for f in range(D // L):
            buf[i, pl.ds(f * L, L)] = z16


def _sc_deg_body(row_hbm, col_hbm, sd_hbm, degp_hbm, gidx_hbm,
                 rowb_v, colb_v, sdb_v, wb_v, gixb_v, zrow_v, deg_sh, sem):
    cid = lax.axis_index("c")
    sid = lax.axis_index("s")
    wid = _wid()

    zl = jnp.zeros((L,), F32)

    @pl.loop(0, CH // L)
    def _zz(i):
        zrow_v[pl.ds(i * L, L)] = zl

    @pl.loop(0, (NROW_CHP - sid + NS - 1) // NS)
    def _zd(k):
        pltpu.sync_copy(zrow_v, deg_sh.at[pl.ds((k * NS + sid) * CH, CH)])
    plsc.subcore_barrier()

    start = jnp.where(wid < 25, wid * 80, 2000 + (wid - 25) * 72)

    def _do_phase(base, n_ph):
        pltpu.sync_copy(row_hbm.at[pl.ds(base, n_ph)], rowb_v.at[pl.ds(0, n_ph)])
        pltpu.sync_copy(col_hbm.at[pl.ds(base, n_ph)], colb_v.at[pl.ds(0, n_ph)])
        pltpu.sync_copy(sd_hbm.at[pl.ds(base, n_ph)], sdb_v.at[pl.ds(0, n_ph)])

        @pl.loop(0, n_ph)
        def _cw(g):
            for i in range(CH // L):
                s = sdb_v[g, pl.ds(i * L, L)]
                r = rowb_v[g, pl.ds(i * L, L)]
                wb_v[g, pl.ds(i * L, L)] = jnp.where(s == 1, 1.0, 0.7).astype(F32)
                gixb_v[g, pl.ds(i * L, L)] = r + (1 - s) * N

        pltpu.sync_copy(gixb_v.at[pl.ds(0, n_ph)], gidx_hbm.at[pl.ds(base, n_ph)])

        @pl.loop(0, n_ph)
        def _fire(g):
            pltpu.async_copy(wb_v.at[g], deg_sh.at[colb_v.at[g]], sem, add=True)

        @pl.loop(0, n_ph)
        def _drain(g):
            pltpu.make_async_copy(wb_v.at[g], deg_sh.at[colb_v.at[g]], sem).wait()

    _do_phase(start, 40)

    @pl.when(wid < 25)
    def _ph1a():
        _do_phase(start + 40, 40)

    @pl.when(wid >= 25)
    def _ph1b():
        _do_phase(start + 40, 32)

    plsc.subcore_barrier()

    @pl.loop(0, (NROW_CHP - sid + NS - 1) // NS)
    def _wb(k):
        o = (k * NS + sid) * CH
        pltpu.sync_copy(deg_sh.at[pl.ds(o, CH)], degp_hbm.at[cid, pl.ds(o, CH)])


def _sc_deg(row2, col2, sd2):
    mesh = plsc.VectorSubcoreMesh(core_axis_name="c", subcore_axis_name="s")
    f = pl.kernel(
        _sc_deg_body,
        out_type=[
            jax.ShapeDtypeStruct((NC, NP), F32),
            jax.ShapeDtypeStruct((NCHP, CH), jnp.int32),
        ],
        mesh=mesh,
        scratch_types=[
            pltpu.VMEM((40, CH), jnp.int32),
            pltpu.VMEM((40, CH), jnp.int32),
            pltpu.VMEM((40, CH), jnp.int32),
            pltpu.VMEM((40, CH), F32),
            pltpu.VMEM((40, CH), jnp.int32),
            pltpu.VMEM((CH,), F32),
            pltpu.VMEM_SHARED((NP,), F32),
            pltpu.SemaphoreType.DMA,
        ],
    )
    return f(row2, col2, sd2)


CHA = 64
NCHPA = NCHP * CH // CHA


def _sc_agg_body(u_hbm, gidx_hbm, col_hbm, p_hbm,
                 gixb_v, colb_v, rows_v, agg_sh, gsem, ssem):
    cid = lax.axis_index("c")
    sid = lax.axis_index("s")
    wid = _wid()

    _zero_vmem(rows_v.at[0], CHA)

    @pl.loop(0, 9)
    def _zs(k):
        pltpu.sync_copy(rows_v.at[0], agg_sh.at[pl.ds(sid * 624 + k * 64, 64)])
    pltpu.sync_copy(rows_v.at[0, pl.ds(0, 48)], agg_sh.at[pl.ds(sid * 624 + 576, 48)])

    @pl.when(sid == 0)
    def _zt():
        pltpu.sync_copy(rows_v.at[0, pl.ds(0, 32)], agg_sh.at[pl.ds(9984, 32)])
    plsc.subcore_barrier()

    start = jnp.where(wid < 18, wid * 160, 2880 + (wid - 18) * 152)

    def _do_phase(base, n_ph):
        pltpu.sync_copy(gidx_hbm.at[pl.ds(base, n_ph)], gixb_v.at[pl.ds(0, n_ph)])
        pltpu.sync_copy(col_hbm.at[pl.ds(base, n_ph)], colb_v.at[pl.ds(0, n_ph)])

        def _fire(k):
            pltpu.async_copy(u_hbm.at[gixb_v.at[k]], rows_v.at[k % 4],
                             gsem.at[k % 4])

        _fire(0)
        _fire(1)

        @pl.loop(0, n_ph)
        def _chunk(g):
            @pl.when(g >= 2)
            def _free():
                p2 = (g - 2) % 4
                pltpu.make_async_copy(rows_v.at[p2],
                                      agg_sh.at[colb_v.at[g - 2]],
                                      ssem.at[p2]).wait()

            @pl.when(g + 2 < n_ph)
            def _next():
                _fire(g + 2)

            p = g % 4
            pltpu.make_async_copy(u_hbm.at[gixb_v.at[g]], rows_v.at[p],
                                  gsem.at[p]).wait()
            pltpu.async_copy(rows_v.at[p], agg_sh.at[colb_v.at[g]],
                             ssem.at[p], add=True)

        for t in (2, 1):
            p2 = (n_ph - t) % 4
            pltpu.make_async_copy(rows_v.at[p2],
                                  agg_sh.at[colb_v.at[n_ph - t]],
                                  ssem.at[p2]).wait()

    for i in range(3):
        _do_phase(start + i * 40, 40)

    @pl.when(wid < 18)
    def _ph1a():
        _do_phase(start + 120, 40)

    @pl.when(wid >= 18)
    def _ph1b():
        _do_phase(start + 120, 32)

    plsc.subcore_barrier()

    @pl.loop(0, 4)
    def _wb(k):
        o = sid * 624 + k * 128
        pltpu.sync_copy(agg_sh.at[pl.ds(o, 128)], p_hbm.at[cid, pl.ds(o, 128)])
    pltpu.sync_copy(agg_sh.at[pl.ds(sid * 624 + 512, 112)],
                    p_hbm.at[cid, pl.ds(sid * 624 + 512, 112)])

    @pl.when(sid == 0)
    def _wt():
        pltpu.sync_copy(agg_sh.at[pl.ds(9984, 16)], p_hbm.at[cid, pl.ds(9984, 16)])


def _sc_agg(u2, gidx2, col2):
    mesh = plsc.VectorSubcoreMesh(core_axis_name="c", subcore_axis_name="s")
    f = pl.kernel(
        _sc_agg_body,
        out_type=[jax.ShapeDtypeStruct((NC, N, D), F32)],
        mesh=mesh,
        scratch_types=[
            pltpu.VMEM((40, CHA), jnp.int32),
            pltpu.VMEM((40, CHA), jnp.int32),
            pltpu.VMEM((4, CHA, D), F32),
            pltpu.VMEM_SHARED((NAGG, D), F32),
            pltpu.SemaphoreType.DMA((4,)),
            pltpu.SemaphoreType.DMA((4,)),
        ],
    )
    return f(u2, gidx2.reshape(NCHPA, CHA), col2.reshape(NCHPA, CHA))[0]


def _sc_pool_body(agg_hbm, ac_hbm, batch_hbm, sums_hbm, cnt_hbm, maxp_hbm,
                  b_v, rows_v, a_v, c_v, one_v, maxb_v, smem_s,
                  sums_sh, cnt_sh, stage_sh, sem):
    cid = lax.axis_index("c")
    sid = lax.axis_index("s")
    wid = _wid()

    ninf = jnp.full((L,), -jnp.inf, F32)

    @pl.loop(0, G + 8)
    def _mi(i):
        for f in range(D // L):
            maxb_v[i, pl.ds(f * L, L)] = ninf

    pltpu.sync_copy(ac_hbm.at[0], a_v)
    pltpu.sync_copy(ac_hbm.at[1], c_v)
    ones = jnp.ones((L,), F32)

    @pl.loop(0, CH // L)
    def _o(i):
        one_v[pl.ds(i * L, L)] = ones

    z16 = jnp.zeros((L,), F32)
    for f in range(D // L):
        rows_v[0, pl.ds(f * L, L)] = z16

    @pl.loop(0, (G + 8 - sid + NS - 1) // NS)
    def _zs(k):
        pltpu.sync_copy(rows_v.at[0], sums_sh.at[k * NS + sid])

    @pl.when(sid < (G + 16) // L)
    def _zc():
        pltpu.sync_copy(rows_v.at[0, pl.ds(0, L)], cnt_sh.at[pl.ds(sid * L, L)])
    plsc.subcore_barrier()

    def _process(off, m):
        pltpu.sync_copy(agg_hbm.at[pl.ds(off, m)], rows_v.at[pl.ds(0, m)])
        pltpu.sync_copy(batch_hbm.at[pl.ds(off, CH)], b_v)
        pltpu.sync_copy(b_v, stage_sh.at[wid])
        pltpu.sync_copy(stage_sh.at[wid], smem_s)

        @pl.loop(0, m)
        def _h(i):
            for f in range(D // L):
                s = pl.ds(f * L, L)
                rows_v[i, s] = jnp.maximum(rows_v[i, s] * a_v[s] + c_v[s], 0.0)

        pltpu.sync_copy(rows_v, sums_sh.at[b_v], add=True)
        pltpu.sync_copy(one_v, cnt_sh.at[b_v], add=True)

        @pl.loop(0, m)
        def _mx(i):
            b = smem_s[i]
            for f in range(D // L):
                s = pl.ds(f * L, L)
                maxb_v[b, s] = jnp.maximum(maxb_v[b, s], rows_v[i, s])

    @pl.loop(0, (NROW_CH - wid + NW - 1) // NW)
    def _chunk(g):
        _process((g * NW + wid) * CH, CH)

    @pl.when(wid == 0)
    def _tail():
        _process(NROW_CH * CH, 16)

    plsc.subcore_barrier()
    pltpu.sync_copy(sums_sh.at[pl.ds(sid * 8, 8)], sums_hbm.at[cid, pl.ds(sid * 8, 8)])

    @pl.when(sid == 0)
    def _wc():
        pltpu.sync_copy(cnt_sh.at[pl.ds(0, G)], cnt_hbm.at[cid])
    pltpu.sync_copy(maxb_v.at[pl.ds(0, G)], maxp_hbm.at[wid])


def _sc_pool(agg3, ac3, batch):
    mesh = plsc.VectorSubcoreMesh(core_axis_name="c", subcore_axis_name="s")
    f = pl.kernel(
        _sc_pool_body,
        out_type=[
            jax.ShapeDtypeStruct((NC, G, D), F32),
            jax.ShapeDtypeStruct((NC, G), F32),
            jax.ShapeDtypeStruct((NW, G, D), F32),
        ],
        mesh=mesh,
        scratch_types=[
            pltpu.VMEM((CH,), jnp.int32),
            pltpu.VMEM((CH, D), F32),
            pltpu.VMEM((D,), F32),
            pltpu.VMEM((D,), F32),
            pltpu.VMEM((CH,), F32),
            pltpu.VMEM((G + 8, D), F32),
            pltpu.SMEM((CH,), jnp.int32),
            pltpu.VMEM_SHARED((G + 8, D), F32),
            pltpu.VMEM_SHARED((G + 16,), F32),
            pltpu.VMEM_SHARED((NW, CH), jnp.int32),
            pltpu.SemaphoreType.DMA,
        ],
    )
    return f(agg3, ac3, batch)


def _dot(a, b):
    return lax.dot_general(a, b, (((1,), (0,)), ((), ())),
                           precision=lax.Precision.HIGHEST,
                           preferred_element_type=F32)


def _p0_body(x_ref, degp_ref, w_ref, z_ref, u_ref):
    deg = degp_ref[0, 0] + degp_ref[0, 1] + 1.0
    dis = lax.rsqrt(deg)
    z = _dot(x_ref[...], w_ref[...])
    z_ref[...] = z
    u = dis[:, None] * z
    u_ref[0] = u
    u_ref[1] = 0.7 * u


def _pl_body(agg_ref, degp_ref, ac_ref, w_ref, zo_ref, uo_ref):
    deg = degp_ref[0, 0] + degp_ref[0, 1] + 1.0
    dis = lax.rsqrt(deg)
    h = jnp.maximum(agg_ref[...] * ac_ref[0, :][None, :] + ac_ref[1, :][None, :], 0.0)
    z = _dot(h, w_ref[...])
    zo_ref[...] = z
    u = dis[:, None] * z
    uo_ref[0] = u
    uo_ref[1] = 0.7 * u


_row_spec = pl.BlockSpec((BLK, D), lambda i: (i, 0))
_degp_spec = pl.BlockSpec((1, NC, BLK), lambda i: (i, 0, 0))
_p_spec = pl.BlockSpec((NC, BLK, D), lambda i: (0, i, 0))
_full_mat = pl.BlockSpec((D, D), lambda i: (0, 0))
_ac_spec = pl.BlockSpec((2, D), lambda i: (0, 0))
_u_spec = pl.BlockSpec((NC, BLK, D), lambda i: (0, i, 0))


def _tc_p0(x, degp, W):
    return pl.pallas_call(
        _p0_body,
        grid=(NBLK,),
        in_specs=[_row_spec, _degp_spec, _full_mat],
        out_specs=[_row_spec, _u_spec],
        out_shape=[
            jax.ShapeDtypeStruct((N, D), F32),
            jax.ShapeDtypeStruct((NC, N, D), F32),
        ],
    )(x, degp, W)


def _tc_pl(agg, degp, ac, W):
    return pl.pallas_call(
        _pl_body,
        grid=(NBLK,),
        in_specs=[_row_spec, _degp_spec, _ac_spec, _full_mat],
        out_specs=[_row_spec, _u_spec],
        out_shape=[
            jax.ShapeDtypeStruct((N, D), F32),
            jax.ShapeDtypeStruct((NC, N, D), F32),
        ],
    )(agg, degp, ac, W)


def _s_body(p_ref, z_ref, degp_ref, b_ref, g_ref, be_ref,
            agg_ref, ac_ref, s1, s2):
    i = pl.program_id(0)

    @pl.when(i == 0)
    def _init():
        s1[...] = jnp.zeros((1, D), F32)
        s2[...] = jnp.zeros((1, D), F32)

    deg = degp_ref[0, 0] + degp_ref[0, 1] + 1.0
    dis = lax.rsqrt(deg)
    agg = (dis[:, None] * (p_ref[0] + p_ref[1])
           + (1.0 / deg)[:, None] * z_ref[...] + b_ref[0, :][None, :])
    agg_ref[...] = agg
    s1[...] += jnp.sum(agg, axis=0, keepdims=True)
    s2[...] += jnp.sum(agg * agg, axis=0, keepdims=True)

    @pl.when(i == NBLK - 1)
    def _fin():
        m = s1[...] / N
        var = s2[...] / N - m * m
        a = g_ref[0, :][None, :] * lax.rsqrt(var + EPS)
        ac_ref[0] = a[0]
        ac_ref[1] = be_ref[0, :] - m[0] * a[0]


_vec_spec = pl.BlockSpec((1, D), lambda i: (0, 0))


def _tc_s(p, z, degp, b, g, be):
    return pl.pallas_call(
        _s_body,
        grid=(NBLK,),
        in_specs=[_p_spec, _row_spec, _degp_spec, _vec_spec, _vec_spec, _vec_spec],
        out_specs=[_row_spec, _ac_spec],
        out_shape=[
            jax.ShapeDtypeStruct((N, D), F32),
            jax.ShapeDtypeStruct((2, D), F32),
        ],
        scratch_shapes=[pltpu.VMEM((1, D), F32), pltpu.VMEM((1, D), F32)],
    )(p, z, degp, b.reshape(1, D), g.reshape(1, D), be.reshape(1, D))


def _head_body(sums_ref, cnt_ref, maxp_ref, w_ref, b_ref, out_ref):
    sums = sums_ref[0] + sums_ref[1]
    cnt = cnt_ref[0, :] + cnt_ref[1, :]
    mean = sums / jnp.maximum(cnt, 1.0)[:, None]
    xmax = jnp.max(maxp_ref[...], axis=0)
    hm = _dot(mean, w_ref[pl.ds(0, D), :])
    hx = _dot(xmax, w_ref[pl.ds(D, D), :])
    logits = hm + hx + b_ref[0, :][None, :]
    mx = jnp.max(logits, axis=1, keepdims=True)
    lse = mx + jnp.log(jnp.sum(jnp.exp(logits - mx), axis=1, keepdims=True))
    out_ref[...] = logits - lse


def _tc_head(sums, cnt, maxp, linW, linb):
    return pl.pallas_call(
        _head_body,
        out_shape=jax.ShapeDtypeStruct((G, 2), F32),
    )(sums, cnt, maxp, linW, linb.reshape(1, 2))


def kernel(x, edge_index, batch, same_date, W0, b0, W1, b1, W2, b2, W3, b3,
           g0, be0, g1, be1, g2, be2, g3, be3, linW, linb):
    npad = NCHP * CH - E
    row2 = jnp.concatenate([edge_index[0], jnp.zeros((npad,), jnp.int32)]).reshape(NCHP, CH)
    col2 = jnp.concatenate([edge_index[1], jnp.full((npad,), DUMMY, jnp.int32)]).reshape(NCHP, CH)
    sd2 = jnp.concatenate([same_date, jnp.ones((npad,), jnp.int32)]).reshape(NCHP, CH)
    degp, gidx2 = _sc_deg(row2, col2, sd2)
    degp = degp[:, :N].reshape(NC, NBLK, BLK).transpose(1, 0, 2)

    layers = [(W0, b0, g0, be0), (W1, b1, g1, be1),
              (W2, b2, g2, be2), (W3, b3, g3, be3)]

    z, u = _tc_p0(x, degp, W0)
    for li in range(4):
        _, b, g, be = layers[li]
        p = _sc_agg(u.reshape(2 * N, D), gidx2, col2)
        agg, ac = _tc_s(p, z, degp, b, g, be)
        if li < 3:
            Wn = layers[li + 1][0]
            z, u = _tc_pl(agg, degp, ac, Wn)
    batch_p = jnp.pad(batch, (0, NP - N), constant_values=G)
    sums, cnt, maxp = _sc_pool(agg, ac, batch_p)
    return _tc_head(sums, cnt, maxp, linW, linb)


# speedup vs baseline: 1.0312x; 1.0312x over previous
import functools
import jax
import jax.numpy as jnp
from jax import lax
from jax.experimental import pallas as pl
from jax.experimental.pallas import tpu as pltpu, tpu_sc as plsc

N = 10000
E = 320000
G = 128
D = 128
NC, NS, L = 2, 16, 16
NW = NC * NS
CH = 128
NCHUNK = E // CH
NROW_CH = N // CH
NP = 10112
NROW_CHP = NP // CH
NCHP = 2504
NAGG = N + 16
DUMMY = N + 8
BLK = 400
NBLK = N // BLK
EPS = 1e-5
F32 = jnp.float32


def _wid():
    return lax.axis_index("s") * NC + lax.axis_index("c")


def _zero_vmem(buf, rows):
    z16 = jnp.zeros((L,), F32)

    @pl.loop(0, rows)
    def _z(i):
        for f in range(D // L):
            buf[i, pl.ds(f * L, L)] = z16


def _sc_deg_body(row_hbm, col_hbm, sd_hbm, degp_hbm, gidx_hbm,
                 rowb_v, colb_v, sdb_v, wb_v, gixb_v, zrow_v, deg_sh, sem):
    cid = lax.axis_index("c")
    sid = lax.axis_index("s")
    wid = _wid()

    zl = jnp.zeros((L,), F32)

    @pl.loop(0, CH // L)
    def _zz(i):
        zrow_v[pl.ds(i * L, L)] = zl

    @pl.loop(0, (NROW_CHP - sid + NS - 1) // NS)
    def _zd(k):
        pltpu.sync_copy(zrow_v, deg_sh.at[pl.ds((k * NS + sid) * CH, CH)])
    plsc.subcore_barrier()

    start = jnp.where(wid < 25, wid * 80, 2000 + (wid - 25) * 72)

    def _do_phase(base, n_ph):
        pltpu.sync_copy(row_hbm.at[pl.ds(base, n_ph)], rowb_v.at[pl.ds(0, n_ph)])
        pltpu.sync_copy(col_hbm.at[pl.ds(base, n_ph)], colb_v.at[pl.ds(0, n_ph)])
        pltpu.sync_copy(sd_hbm.at[pl.ds(base, n_ph)], sdb_v.at[pl.ds(0, n_ph)])

        @pl.loop(0, n_ph)
        def _cw(g):
            for i in range(CH // L):
                s = sdb_v[g, pl.ds(i * L, L)]
                r = rowb_v[g, pl.ds(i * L, L)]
                wb_v[g, pl.ds(i * L, L)] = jnp.where(s == 1, 1.0, 0.7).astype(F32)
                gixb_v[g, pl.ds(i * L, L)] = r + (1 - s) * N

        pltpu.sync_copy(gixb_v.at[pl.ds(0, n_ph)], gidx_hbm.at[pl.ds(base, n_ph)])

        @pl.loop(0, n_ph)
        def _fire(g):
            pltpu.async_copy(wb_v.at[g], deg_sh.at[colb_v.at[g]], sem, add=True)

        @pl.loop(0, n_ph)
        def _drain(g):
            pltpu.make_async_copy(wb_v.at[g], deg_sh.at[colb_v.at[g]], sem).wait()

    _do_phase(start, 40)

    @pl.when(wid < 25)
    def _ph1a():
        _do_phase(start + 40, 40)

    @pl.when(wid >= 25)
    def _ph1b():
        _do_phase(start + 40, 32)

    plsc.subcore_barrier()

    @pl.loop(0, (NROW_CHP - sid + NS - 1) // NS)
    def _wb(k):
        o = (k * NS + sid) * CH
        pltpu.sync_copy(deg_sh.at[pl.ds(o, CH)], degp_hbm.at[cid, pl.ds(o, CH)])


def _sc_deg(row2, col2, sd2):
    mesh = plsc.VectorSubcoreMesh(core_axis_name="c", subcore_axis_name="s")
    f = pl.kernel(
        _sc_deg_body,
        out_type=[
            jax.ShapeDtypeStruct((NC, NP), F32),
            jax.ShapeDtypeStruct((NCHP, CH), jnp.int32),
        ],
        mesh=mesh,
        scratch_types=[
            pltpu.VMEM((40, CH), jnp.int32),
            pltpu.VMEM((40, CH), jnp.int32),
            pltpu.VMEM((40, CH), jnp.int32),
            pltpu.VMEM((40, CH), F32),
            pltpu.VMEM((40, CH), jnp.int32),
            pltpu.VMEM((CH,), F32),
            pltpu.VMEM_SHARED((NP,), F32),
            pltpu.SemaphoreType.DMA,
        ],
    )
    return f(row2, col2, sd2)


def _sc_agg_body(u_hbm, gidx_hbm, col_hbm, p_hbm,
                 gixb_v, colb_v, rows_v, agg_sh, gsem):
    cid = lax.axis_index("c")
    sid = lax.axis_index("s")
    wid = _wid()

    _zero_vmem(rows_v.at[0], CH)

    @pl.loop(0, 4)
    def _zs(k):
        pltpu.sync_copy(rows_v.at[0], agg_sh.at[pl.ds(sid * 624 + k * 128, 128)])
    pltpu.sync_copy(rows_v.at[0, pl.ds(0, 112)], agg_sh.at[pl.ds(sid * 624 + 512, 112)])

    @pl.when(sid == 0)
    def _zt():
        pltpu.sync_copy(rows_v.at[0, pl.ds(0, 32)], agg_sh.at[pl.ds(9984, 32)])
    plsc.subcore_barrier()

    start = jnp.where(wid < 25, wid * 80, 2000 + (wid - 25) * 72)

    def _fire(k):
        pltpu.async_copy(u_hbm.at[gixb_v.at[k]], rows_v.at[k % 2],
                         gsem.at[k % 2])

    def _do_phase(base, n_ph):
        pltpu.sync_copy(gidx_hbm.at[pl.ds(base, n_ph)], gixb_v.at[pl.ds(0, n_ph)])
        pltpu.sync_copy(col_hbm.at[pl.ds(base, n_ph)], colb_v.at[pl.ds(0, n_ph)])
        _fire(0)
        _fire(1)

        @pl.loop(0, n_ph)
        def _chunk(g):
            p = g % 2
            pltpu.make_async_copy(u_hbm.at[gixb_v.at[g]], rows_v.at[p],
                                  gsem.at[p]).wait()
            pltpu.sync_copy(rows_v.at[p], agg_sh.at[colb_v.at[g]], add=True)

            @pl.when(g + 2 < n_ph)
            def _next():
                _fire(g + 2)

    _do_phase(start, 40)

    @pl.when(wid < 25)
    def _ph1a():
        _do_phase(start + 40, 40)

    @pl.when(wid >= 25)
    def _ph1b():
        _do_phase(start + 40, 32)

    plsc.subcore_barrier()

    @pl.loop(0, 4)
    def _wb(k):
        o = sid * 624 + k * 128
        pltpu.sync_copy(agg_sh.at[pl.ds(o, 128)], p_hbm.at[cid, pl.ds(o, 128)])
    pltpu.sync_copy(agg_sh.at[pl.ds(sid * 624 + 512, 112)],
                    p_hbm.at[cid, pl.ds(sid * 624 + 512, 112)])

    @pl.when(sid == 0)
    def _wt():
        pltpu.sync_copy(agg_sh.at[pl.ds(9984, 16)], p_hbm.at[cid, pl.ds(9984, 16)])


def _sc_agg(u2, gidx2, col2):
    mesh = plsc.VectorSubcoreMesh(core_axis_name="c", subcore_axis_name="s")
    f = pl.kernel(
        _sc_agg_body,
        out_type=[jax.ShapeDtypeStruct((NC, N, D), F32)],
        mesh=mesh,
        scratch_types=[
            pltpu.VMEM((40, CH), jnp.int32),
            pltpu.VMEM((40, CH), jnp.int32),
            pltpu.VMEM((2, CH, D), F32),
            pltpu.VMEM_SHARED((NAGG, D), F32),
            pltpu.SemaphoreType.DMA((2,)),
        ],
    )
    return f(u2, gidx2, col2)[0]


def _sc_pool_body(agg_hbm, ac_hbm, batch_hbm, sums_hbm, cnt_hbm, maxp_hbm,
                  b_v, rows_v, a_v, c_v, one_v, maxb_v, smem_s,
                  sums_sh, cnt_sh, stage_sh, sem):
    cid = lax.axis_index("c")
    sid = lax.axis_index("s")
    wid = _wid()

    ninf = jnp.full((L,), -jnp.inf, F32)

    @pl.loop(0, G + 8)
    def _mi(i):
        for f in range(D // L):
            maxb_v[i, pl.ds(f * L, L)] = ninf

    pltpu.sync_copy(ac_hbm.at[0], a_v)
    pltpu.sync_copy(ac_hbm.at[1], c_v)
    ones = jnp.ones((L,), F32)

    @pl.loop(0, CH // L)
    def _o(i):
        one_v[pl.ds(i * L, L)] = ones

    z16 = jnp.zeros((L,), F32)
    for f in range(D // L):
        rows_v[0, pl.ds(f * L, L)] = z16

    @pl.loop(0, (G + 8 - sid + NS - 1) // NS)
    def _zs(k):
        pltpu.sync_copy(rows_v.at[0], sums_sh.at[k * NS + sid])

    @pl.when(sid < (G + 16) // L)
    def _zc():
        pltpu.sync_copy(rows_v.at[0, pl.ds(0, L)], cnt_sh.at[pl.ds(sid * L, L)])
    plsc.subcore_barrier()

    def _process(off, m):
        pltpu.sync_copy(agg_hbm.at[pl.ds(off, m)], rows_v.at[pl.ds(0, m)])
        pltpu.sync_copy(batch_hbm.at[pl.ds(off, CH)], b_v)
        pltpu.sync_copy(b_v, stage_sh.at[wid])
        pltpu.sync_copy(stage_sh.at[wid], smem_s)

        @pl.loop(0, m)
        def _h(i):
            for f in range(D // L):
                s = pl.ds(f * L, L)
                rows_v[i, s] = jnp.maximum(rows_v[i, s] * a_v[s] + c_v[s], 0.0)

        pltpu.sync_copy(rows_v, sums_sh.at[b_v], add=True)
        pltpu.sync_copy(one_v, cnt_sh.at[b_v], add=True)

        @pl.loop(0, m)
        def _mx(i):
            b = smem_s[i]
            for f in range(D // L):
                s = pl.ds(f * L, L)
                maxb_v[b, s] = jnp.maximum(maxb_v[b, s], rows_v[i, s])

    @pl.loop(0, (NROW_CH - wid + NW - 1) // NW)
    def _chunk(g):
        _process((g * NW + wid) * CH, CH)

    @pl.when(wid == 0)
    def _tail():
        _process(NROW_CH * CH, 16)

    plsc.subcore_barrier()
    pltpu.sync_copy(sums_sh.at[pl.ds(sid * 8, 8)], sums_hbm.at[cid, pl.ds(sid * 8, 8)])

    @pl.when(sid == 0)
    def _wc():
        pltpu.sync_copy(cnt_sh.at[pl.ds(0, G)], cnt_hbm.at[cid])
    pltpu.sync_copy(maxb_v.at[pl.ds(0, G)], maxp_hbm.at[wid])


def _sc_pool(agg3, ac3, batch):
    mesh = plsc.VectorSubcoreMesh(core_axis_name="c", subcore_axis_name="s")
    f = pl.kernel(
        _sc_pool_body,
        out_type=[
            jax.ShapeDtypeStruct((NC, G, D), F32),
            jax.ShapeDtypeStruct((NC, G), F32),
            jax.ShapeDtypeStruct((NW, G, D), F32),
        ],
        mesh=mesh,
        scratch_types=[
            pltpu.VMEM((CH,), jnp.int32),
            pltpu.VMEM((CH, D), F32),
            pltpu.VMEM((D,), F32),
            pltpu.VMEM((D,), F32),
            pltpu.VMEM((CH,), F32),
            pltpu.VMEM((G + 8, D), F32),
            pltpu.SMEM((CH,), jnp.int32),
            pltpu.VMEM_SHARED((G + 8, D), F32),
            pltpu.VMEM_SHARED((G + 16,), F32),
            pltpu.VMEM_SHARED((NW, CH), jnp.int32),
            pltpu.SemaphoreType.DMA,
        ],
    )
    return f(agg3, ac3, batch)


def _dot(a, b):
    return lax.dot_general(a, b, (((1,), (0,)), ((), ())),
                           precision=lax.Precision.HIGHEST,
                           preferred_element_type=F32)


def _p0_body(x_ref, degp_ref, w_ref, z_ref, u_ref):
    deg = degp_ref[0, 0] + degp_ref[0, 1] + 1.0
    dis = lax.rsqrt(deg)
    z = _dot(x_ref[...], w_ref[...])
    z_ref[...] = z
    u = dis[:, None] * z
    u_ref[0] = u
    u_ref[1] = 0.7 * u


def _pl_body(agg_ref, degp_ref, ac_ref, w_ref, zo_ref, uo_ref):
    deg = degp_ref[0, 0] + degp_ref[0, 1] + 1.0
    dis = lax.rsqrt(deg)
    h = jnp.maximum(agg_ref[...] * ac_ref[0, :][None, :] + ac_ref[1, :][None, :], 0.0)
    z = _dot(h, w_ref[...])
    zo_ref[...] = z
    u = dis[:, None] * z
    uo_ref[0] = u
    uo_ref[1] = 0.7 * u


_row_spec = pl.BlockSpec((BLK, D), lambda i: (i, 0))
_degp_spec = pl.BlockSpec((1, NC, BLK), lambda i: (i, 0, 0))
_p_spec = pl.BlockSpec((NC, BLK, D), lambda i: (0, i, 0))
_full_mat = pl.BlockSpec((D, D), lambda i: (0, 0))
_ac_spec = pl.BlockSpec((2, D), lambda i: (0, 0))
_u_spec = pl.BlockSpec((NC, BLK, D), lambda i: (0, i, 0))


def _tc_p0(x, degp, W):
    return pl.pallas_call(
        _p0_body,
        grid=(NBLK,),
        in_specs=[_row_spec, _degp_spec, _full_mat],
        out_specs=[_row_spec, _u_spec],
        out_shape=[
            jax.ShapeDtypeStruct((N, D), F32),
            jax.ShapeDtypeStruct((NC, N, D), F32),
        ],
    )(x, degp, W)


def _tc_pl(agg, degp, ac, W):
    return pl.pallas_call(
        _pl_body,
        grid=(NBLK,),
        in_specs=[_row_spec, _degp_spec, _ac_spec, _full_mat],
        out_specs=[_row_spec, _u_spec],
        out_shape=[
            jax.ShapeDtypeStruct((N, D), F32),
            jax.ShapeDtypeStruct((NC, N, D), F32),
        ],
    )(agg, degp, ac, W)


def _s_body(p_ref, z_ref, degp_ref, b_ref, g_ref, be_ref,
            agg_ref, ac_ref, s1, s2):
    i = pl.program_id(0)

    @pl.when(i == 0)
    def _init():
        s1[...] = jnp.zeros((1, D), F32)
        s2[...] = jnp.zeros((1, D), F32)

    deg = degp_ref[0, 0] + degp_ref[0, 1] + 1.0
    dis = lax.rsqrt(deg)
    agg = (dis[:, None] * (p_ref[0] + p_ref[1])
           + (1.0 / deg)[:, None] * z_ref[...] + b_ref[0, :][None, :])
    agg_ref[...] = agg
    s1[...] += jnp.sum(agg, axis=0, keepdims=True)
    s2[...] += jnp.sum(agg * agg, axis=0, keepdims=True)

    @pl.when(i == NBLK - 1)
    def _fin():
        m = s1[...] / N
        var = s2[...] / N - m * m
        a = g_ref[0, :][None, :] * lax.rsqrt(var + EPS)
        ac_ref[0] = a[0]
        ac_ref[1] = be_ref[0, :] - m[0] * a[0]


_vec_spec = pl.BlockSpec((1, D), lambda i: (0, 0))


def _tc_s(p, z, degp, b, g, be):
    return pl.pallas_call(
        _s_body,
        grid=(NBLK,),
        in_specs=[_p_spec, _row_spec, _degp_spec, _vec_spec, _vec_spec, _vec_spec],
        out_specs=[_row_spec, _ac_spec],
        out_shape=[
            jax.ShapeDtypeStruct((N, D), F32),
            jax.ShapeDtypeStruct((2, D), F32),
        ],
        scratch_shapes=[pltpu.VMEM((1, D), F32), pltpu.VMEM((1, D), F32)],
    )(p, z, degp, b.reshape(1, D), g.reshape(1, D), be.reshape(1, D))


def _head_body(sums_ref, cnt_ref, maxp_ref, w_ref, b_ref, out_ref):
    sums = sums_ref[0] + sums_ref[1]
    cnt = cnt_ref[0, :] + cnt_ref[1, :]
    mean = sums / jnp.maximum(cnt, 1.0)[:, None]
    xmax = jnp.max(maxp_ref[...], axis=0)
    hm = _dot(mean, w_ref[pl.ds(0, D), :])
    hx = _dot(xmax, w_ref[pl.ds(D, D), :])
    logits = hm + hx + b_ref[0, :][None, :]
    mx = jnp.max(logits, axis=1, keepdims=True)
    lse = mx + jnp.log(jnp.sum(jnp.exp(logits - mx), axis=1, keepdims=True))
    out_ref[...] = logits - lse


def _tc_head(sums, cnt, maxp, linW, linb):
    return pl.pallas_call(
        _head_body,
        out_shape=jax.ShapeDtypeStruct((G, 2), F32),
    )(sums, cnt, maxp, linW, linb.reshape(1, 2))


def kernel(x, edge_index, batch, same_date, W0, b0, W1, b1, W2, b2, W3, b3,
           g0, be0, g1, be1, g2, be2, g3, be3, linW, linb):
    npad = NCHP * CH - E
    row2 = jnp.concatenate([edge_index[0], jnp.zeros((npad,), jnp.int32)]).reshape(NCHP, CH)
    col2 = jnp.concatenate([edge_index[1], jnp.full((npad,), DUMMY, jnp.int32)]).reshape(NCHP, CH)
    sd2 = jnp.concatenate([same_date, jnp.ones((npad,), jnp.int32)]).reshape(NCHP, CH)
    degp, gidx2 = _sc_deg(row2, col2, sd2)
    degp = degp[:, :N].reshape(NC, NBLK, BLK).transpose(1, 0, 2)

    layers = [(W0, b0, g0, be0), (W1, b1, g1, be1),
              (W2, b2, g2, be2), (W3, b3, g3, be3)]

    z, u = _tc_p0(x, degp, W0)
    for li in range(4):
        _, b, g, be = layers[li]
        p = _sc_agg(u.reshape(2 * N, D), gidx2, col2)
        agg, ac = _tc_s(p, z, degp, b, g, be)
        if li < 3:
            Wn = layers[li + 1][0]
            z, u = _tc_pl(agg, degp, ac, Wn)
    batch_p = jnp.pad(batch, (0, NP - N), constant_values=G)
    sums, cnt, maxp = _sc_pool(agg, ac, batch_p)
    return _tc_head(sums, cnt, maxp, linW, linb)


# speedup vs baseline: 1.0553x; 1.0234x over previous
import functools
import jax
import jax.numpy as jnp
from jax import lax
from jax.experimental import pallas as pl
from jax.experimental.pallas import tpu as pltpu, tpu_sc as plsc

N = 10000
E = 320000
G = 128
D = 128
NC, NS, L = 2, 16, 16
NW = NC * NS
CH = 128
NCHUNK = E // CH
NROW_CH = N // CH
NP = 10112
NROW_CHP = NP // CH
NCHP = 2504
NAGG = N + 16
DUMMY = N + 8
BLK = 400
NBLK = N // BLK
EPS = 1e-5
F32 = jnp.float32


def _wid():
    return lax.axis_index("s") * NC + lax.axis_index("c")


def _zero_vmem(buf, rows):
    z16 = jnp.zeros((L,), F32)

    @pl.loop(0, rows)
    def _z(i):
        for f in range(D // L):
            buf[i, pl.ds(f * L, L)] = z16


def _sc_deg_body(row_hbm, col_hbm, sd_hbm, degp_hbm, gidx_hbm,
                 rowb_v, colb_v, sdb_v, wb_v, gixb_v, zrow_v, deg_sh, sem):
    cid = lax.axis_index("c")
    sid = lax.axis_index("s")
    wid = _wid()

    zl = jnp.zeros((L,), F32)

    @pl.loop(0, CH // L)
    def _zz(i):
        zrow_v[pl.ds(i * L, L)] = zl

    @pl.loop(0, (NROW_CHP - sid + NS - 1) // NS)
    def _zd(k):
        pltpu.sync_copy(zrow_v, deg_sh.at[pl.ds((k * NS + sid) * CH, CH)])
    plsc.subcore_barrier()

    start = jnp.where(wid < 25, wid * 80, 2000 + (wid - 25) * 72)

    def _do_phase(base, n_ph):
        pltpu.sync_copy(row_hbm.at[pl.ds(base, n_ph)], rowb_v.at[pl.ds(0, n_ph)])
        pltpu.sync_copy(col_hbm.at[pl.ds(base, n_ph)], colb_v.at[pl.ds(0, n_ph)])
        pltpu.sync_copy(sd_hbm.at[pl.ds(base, n_ph)], sdb_v.at[pl.ds(0, n_ph)])

        @pl.loop(0, n_ph)
        def _cw(g):
            for i in range(CH // L):
                s = sdb_v[g, pl.ds(i * L, L)]
                r = rowb_v[g, pl.ds(i * L, L)]
                wb_v[g, pl.ds(i * L, L)] = jnp.where(s == 1, 1.0, 0.7).astype(F32)
                gixb_v[g, pl.ds(i * L, L)] = r + (1 - s) * N

        pltpu.sync_copy(gixb_v.at[pl.ds(0, n_ph)], gidx_hbm.at[pl.ds(base, n_ph)])

        @pl.loop(0, n_ph)
        def _fire(g):
            pltpu.async_copy(wb_v.at[g], deg_sh.at[colb_v.at[g]], sem, add=True)

        @pl.loop(0, n_ph)
        def _drain(g):
            pltpu.make_async_copy(wb_v.at[g], deg_sh.at[colb_v.at[g]], sem).wait()

    _do_phase(start, 40)

    @pl.when(wid < 25)
    def _ph1a():
        _do_phase(start + 40, 40)

    @pl.when(wid >= 25)
    def _ph1b():
        _do_phase(start + 40, 32)

    plsc.subcore_barrier()

    @pl.loop(0, (NROW_CHP - sid + NS - 1) // NS)
    def _wb(k):
        o = (k * NS + sid) * CH
        pltpu.sync_copy(deg_sh.at[pl.ds(o, CH)], degp_hbm.at[cid, pl.ds(o, CH)])


def _sc_deg(row2, col2, sd2):
    mesh = plsc.VectorSubcoreMesh(core_axis_name="c", subcore_axis_name="s")
    f = pl.kernel(
        _sc_deg_body,
        out_type=[
            jax.ShapeDtypeStruct((NC, NP), F32),
            jax.ShapeDtypeStruct((NCHP, CH), jnp.int32),
        ],
        mesh=mesh,
        scratch_types=[
            pltpu.VMEM((40, CH), jnp.int32),
            pltpu.VMEM((40, CH), jnp.int32),
            pltpu.VMEM((40, CH), jnp.int32),
            pltpu.VMEM((40, CH), F32),
            pltpu.VMEM((40, CH), jnp.int32),
            pltpu.VMEM((CH,), F32),
            pltpu.VMEM_SHARED((NP,), F32),
            pltpu.SemaphoreType.DMA,
        ],
    )
    return f(row2, col2, sd2)


def _sc_agg_body(u_hbm, gidx_hbm, col_hbm, p_hbm,
                 gixb_v, colb_v, rows_v, agg_sh, gsem):
    cid = lax.axis_index("c")
    sid = lax.axis_index("s")
    wid = _wid()

    _zero_vmem(rows_v.at[0], CH)

    @pl.loop(0, 4)
    def _zs(k):
        pltpu.sync_copy(rows_v.at[0], agg_sh.at[pl.ds(sid * 624 + k * 128, 128)])
    pltpu.sync_copy(rows_v.at[0, pl.ds(0, 112)], agg_sh.at[pl.ds(sid * 624 + 512, 112)])

    @pl.when(sid == 0)
    def _zt():
        pltpu.sync_copy(rows_v.at[0, pl.ds(0, 32)], agg_sh.at[pl.ds(9984, 32)])
    plsc.subcore_barrier()

    start = jnp.where(wid < 25, wid * 80, 2000 + (wid - 25) * 72)

    def _fire(k):
        pltpu.async_copy(u_hbm.at[gixb_v.at[k]], rows_v.at[k % 2],
                         gsem.at[k % 2])

    def _do_phase(base, n_ph):
        pltpu.sync_copy(gidx_hbm.at[pl.ds(base, n_ph)], gixb_v.at[pl.ds(0, n_ph)])
        pltpu.sync_copy(col_hbm.at[pl.ds(base, n_ph)], colb_v.at[pl.ds(0, n_ph)])
        _fire(0)
        _fire(1)

        @pl.loop(0, n_ph)
        def _chunk(g):
            p = g % 2
            pltpu.make_async_copy(u_hbm.at[gixb_v.at[g]], rows_v.at[p],
                                  gsem.at[p]).wait()
            pltpu.sync_copy(rows_v.at[p], agg_sh.at[colb_v.at[g]], add=True)

            @pl.when(g + 2 < n_ph)
            def _next():
                _fire(g + 2)

    _do_phase(start, 40)

    @pl.when(wid < 25)
    def _ph1a():
        _do_phase(start + 40, 40)

    @pl.when(wid >= 25)
    def _ph1b():
        _do_phase(start + 40, 32)

    plsc.subcore_barrier()

    @pl.loop(0, 4)
    def _wb(k):
        o = sid * 624 + k * 128
        pltpu.sync_copy(agg_sh.at[pl.ds(o, 128)], p_hbm.at[cid, pl.ds(o, 128)])
    pltpu.sync_copy(agg_sh.at[pl.ds(sid * 624 + 512, 112)],
                    p_hbm.at[cid, pl.ds(sid * 624 + 512, 112)])

    @pl.when(sid == 0)
    def _wt():
        pltpu.sync_copy(agg_sh.at[pl.ds(9984, 16)], p_hbm.at[cid, pl.ds(9984, 16)])


def _sc_agg(u2, gidx2, col2):
    mesh = plsc.VectorSubcoreMesh(core_axis_name="c", subcore_axis_name="s")
    f = pl.kernel(
        _sc_agg_body,
        out_type=[jax.ShapeDtypeStruct((NC, N, D), F32)],
        mesh=mesh,
        scratch_types=[
            pltpu.VMEM((40, CH), jnp.int32),
            pltpu.VMEM((40, CH), jnp.int32),
            pltpu.VMEM((2, CH, D), F32),
            pltpu.VMEM_SHARED((NAGG, D), F32),
            pltpu.SemaphoreType.DMA((2,)),
        ],
    )
    return f(u2, gidx2, col2)[0]


def _sc_pool_body(agg_hbm, ac_hbm, batch_hbm, sums_hbm, cnt_hbm, maxp_hbm,
                  b_v, rows_v, a_v, c_v, one_v, maxb_v, smem_s,
                  sums_sh, cnt_sh, stage_sh, sem):
    cid = lax.axis_index("c")
    sid = lax.axis_index("s")
    wid = _wid()

    ninf = jnp.full((L,), -jnp.inf, F32)

    @pl.loop(0, G + 8)
    def _mi(i):
        for f in range(D // L):
            maxb_v[i, pl.ds(f * L, L)] = ninf

    pltpu.sync_copy(ac_hbm.at[0], a_v)
    pltpu.sync_copy(ac_hbm.at[1], c_v)
    ones = jnp.ones((L,), F32)

    @pl.loop(0, CH // L)
    def _o(i):
        one_v[pl.ds(i * L, L)] = ones

    z16 = jnp.zeros((L,), F32)
    for f in range(D // L):
        rows_v[0, pl.ds(f * L, L)] = z16

    @pl.loop(0, (G + 8 - sid + NS - 1) // NS)
    def _zs(k):
        pltpu.sync_copy(rows_v.at[0], sums_sh.at[k * NS + sid])

    @pl.when(sid < (G + 16) // L)
    def _zc():
        pltpu.sync_copy(rows_v.at[0, pl.ds(0, L)], cnt_sh.at[pl.ds(sid * L, L)])
    plsc.subcore_barrier()

    def _process(off, m):
        pltpu.sync_copy(agg_hbm.at[pl.ds(off, m)], rows_v.at[pl.ds(0, m)])
        pltpu.sync_copy(batch_hbm.at[pl.ds(off, CH)], b_v)
        pltpu.sync_copy(b_v, stage_sh.at[wid])
        pltpu.sync_copy(stage_sh.at[wid], smem_s)

        @pl.loop(0, m)
        def _h(i):
            for f in range(D // L):
                s = pl.ds(f * L, L)
                rows_v[i, s] = jnp.maximum(rows_v[i, s] * a_v[s] + c_v[s], 0.0)

        pltpu.sync_copy(rows_v, sums_sh.at[b_v], add=True)
        pltpu.sync_copy(one_v, cnt_sh.at[b_v], add=True)

        @pl.loop(0, m)
        def _mx(i):
            b = smem_s[i]
            for f in range(D // L):
                s = pl.ds(f * L, L)
                maxb_v[b, s] = jnp.maximum(maxb_v[b, s], rows_v[i, s])

    @pl.loop(0, (NROW_CH - wid + NW - 1) // NW)
    def _chunk(g):
        _process((g * NW + wid) * CH, CH)

    @pl.when(wid == 0)
    def _tail():
        _process(NROW_CH * CH, 16)

    plsc.subcore_barrier()
    pltpu.sync_copy(sums_sh.at[pl.ds(sid * 8, 8)], sums_hbm.at[cid, pl.ds(sid * 8, 8)])

    @pl.when(sid == 0)
    def _wc():
        pltpu.sync_copy(cnt_sh.at[pl.ds(0, G)], cnt_hbm.at[cid])
    pltpu.sync_copy(maxb_v.at[pl.ds(0, G)], maxp_hbm.at[wid])


def _sc_pool(agg3, ac3, batch):
    mesh = plsc.VectorSubcoreMesh(core_axis_name="c", subcore_axis_name="s")
    f = pl.kernel(
        _sc_pool_body,
        out_type=[
            jax.ShapeDtypeStruct((NC, G, D), F32),
            jax.ShapeDtypeStruct((NC, G), F32),
            jax.ShapeDtypeStruct((NW, G, D), F32),
        ],
        mesh=mesh,
        scratch_types=[
            pltpu.VMEM((CH,), jnp.int32),
            pltpu.VMEM((CH, D), F32),
            pltpu.VMEM((D,), F32),
            pltpu.VMEM((D,), F32),
            pltpu.VMEM((CH,), F32),
            pltpu.VMEM((G + 8, D), F32),
            pltpu.SMEM((CH,), jnp.int32),
            pltpu.VMEM_SHARED((G + 8, D), F32),
            pltpu.VMEM_SHARED((G + 16,), F32),
            pltpu.VMEM_SHARED((NW, CH), jnp.int32),
            pltpu.SemaphoreType.DMA,
        ],
    )
    return f(agg3, ac3, batch)


def _dot(a, b):
    return lax.dot_general(a, b, (((1,), (0,)), ((), ())),
                           precision=lax.Precision.HIGHEST,
                           preferred_element_type=F32)


def _p0_body(x_ref, degp_ref, w_ref, z_ref, u_ref):
    deg = degp_ref[0, 0] + degp_ref[0, 1] + 1.0
    dis = lax.rsqrt(deg)
    z = _dot(x_ref[...], w_ref[...])
    z_ref[...] = z
    u = dis[:, None] * z
    u_ref[0] = u
    u_ref[1] = 0.7 * u


def _pl_body(agg_ref, degp_ref, ac_ref, w_ref, zo_ref, uo_ref):
    deg = degp_ref[0, 0] + degp_ref[0, 1] + 1.0
    dis = lax.rsqrt(deg)
    h = jnp.maximum(agg_ref[...] * ac_ref[0, :][None, :] + ac_ref[1, :][None, :], 0.0)
    z = _dot(h, w_ref[...])
    zo_ref[...] = z
    u = dis[:, None] * z
    uo_ref[0] = u
    uo_ref[1] = 0.7 * u


_row_spec = pl.BlockSpec((BLK, D), lambda i: (i, 0))
_degp_spec = pl.BlockSpec((1, NC, BLK), lambda i: (i, 0, 0))
_p_spec = pl.BlockSpec((NC, BLK, D), lambda i: (0, i, 0))
_full_mat = pl.BlockSpec((D, D), lambda i: (0, 0))
_ac_spec = pl.BlockSpec((2, D), lambda i: (0, 0))
_u_spec = pl.BlockSpec((NC, BLK, D), lambda i: (0, i, 0))


def _tc_p0(x, degp, W):
    return pl.pallas_call(
        _p0_body,
        grid=(NBLK,),
        in_specs=[_row_spec, _degp_spec, _full_mat],
        out_specs=[_row_spec, _u_spec],
        out_shape=[
            jax.ShapeDtypeStruct((N, D), F32),
            jax.ShapeDtypeStruct((NC, N, D), F32),
        ],
    )(x, degp, W)


def _tc_pl(agg, degp, ac, W):
    return pl.pallas_call(
        _pl_body,
        grid=(NBLK,),
        in_specs=[_row_spec, _degp_spec, _ac_spec, _full_mat],
        out_specs=[_row_spec, _u_spec],
        out_shape=[
            jax.ShapeDtypeStruct((N, D), F32),
            jax.ShapeDtypeStruct((NC, N, D), F32),
        ],
    )(agg, degp, ac, W)


def _sp_body(p_ref, z_ref, degp_ref, b_ref, g_ref, be_ref, w_ref,
             zo_ref, uo_ref, ac_ref, agg_s, s1, s2):
    i = pl.program_id(0)

    @pl.when(i == 0)
    def _init():
        s1[...] = jnp.zeros((1, D), F32)
        s2[...] = jnp.zeros((1, D), F32)

    deg = degp_ref[0, 0] + degp_ref[0, 1] + 1.0
    dis = lax.rsqrt(deg)

    @pl.when(i < NBLK)
    def _statsphase():
        agg = (dis[:, None] * (p_ref[0] + p_ref[1])
               + (1.0 / deg)[:, None] * z_ref[...] + b_ref[0, :][None, :])
        agg_s[pl.ds(i * BLK, BLK), :] = agg
        s1[...] += jnp.sum(agg, axis=0, keepdims=True)
        s2[...] += jnp.sum(agg * agg, axis=0, keepdims=True)

        @pl.when(i == NBLK - 1)
        def _fin():
            m = s1[...] / N
            var = s2[...] / N - m * m
            a = g_ref[0, :][None, :] * lax.rsqrt(var + EPS)
            ac_ref[0] = a[0]
            ac_ref[1] = be_ref[0, :] - m[0] * a[0]

    @pl.when(i >= NBLK)
    def _applyphase():
        j = i - NBLK
        agg = agg_s[pl.ds(j * BLK, BLK), :]
        h = jnp.maximum(agg * ac_ref[0, :][None, :] + ac_ref[1, :][None, :], 0.0)
        z = _dot(h, w_ref[...])
        zo_ref[...] = z
        u = dis[:, None] * z
        uo_ref[0] = u
        uo_ref[1] = 0.7 * u


def _tc_sp(p, z, degp, b, g, be, Wn):
    iA = lambda i: jnp.minimum(i, NBLK - 1)
    iB = lambda i: jnp.maximum(i - NBLK, 0)
    iAB = lambda i: jnp.where(i < NBLK, i, i - NBLK)
    return pl.pallas_call(
        _sp_body,
        grid=(2 * NBLK,),
        in_specs=[
            pl.BlockSpec((NC, BLK, D), lambda i: (0, jnp.minimum(i, NBLK - 1), 0)),
            pl.BlockSpec((BLK, D), lambda i: (jnp.minimum(i, NBLK - 1), 0)),
            pl.BlockSpec((1, NC, BLK), lambda i: (jnp.where(i < NBLK, i, i - NBLK), 0, 0)),
            _vec_spec, _vec_spec, _vec_spec, _full_mat,
        ],
        out_specs=[
            pl.BlockSpec((BLK, D), lambda i: (jnp.maximum(i - NBLK, 0), 0)),
            pl.BlockSpec((NC, BLK, D), lambda i: (0, jnp.maximum(i - NBLK, 0), 0)),
            _ac_spec,
        ],
        out_shape=[
            jax.ShapeDtypeStruct((N, D), F32),
            jax.ShapeDtypeStruct((NC, N, D), F32),
            jax.ShapeDtypeStruct((2, D), F32),
        ],
        scratch_shapes=[pltpu.VMEM((N, D), F32),
                        pltpu.VMEM((1, D), F32), pltpu.VMEM((1, D), F32)],
    )(p, z, degp, b.reshape(1, D), g.reshape(1, D), be.reshape(1, D), Wn)


def _s_body(p_ref, z_ref, degp_ref, b_ref, g_ref, be_ref,
            agg_ref, ac_ref, s1, s2):
    i = pl.program_id(0)

    @pl.when(i == 0)
    def _init():
        s1[...] = jnp.zeros((1, D), F32)
        s2[...] = jnp.zeros((1, D), F32)

    deg = degp_ref[0, 0] + degp_ref[0, 1] + 1.0
    dis = lax.rsqrt(deg)
    agg = (dis[:, None] * (p_ref[0] + p_ref[1])
           + (1.0 / deg)[:, None] * z_ref[...] + b_ref[0, :][None, :])
    agg_ref[...] = agg
    s1[...] += jnp.sum(agg, axis=0, keepdims=True)
    s2[...] += jnp.sum(agg * agg, axis=0, keepdims=True)

    @pl.when(i == NBLK - 1)
    def _fin():
        m = s1[...] / N
        var = s2[...] / N - m * m
        a = g_ref[0, :][None, :] * lax.rsqrt(var + EPS)
        ac_ref[0] = a[0]
        ac_ref[1] = be_ref[0, :] - m[0] * a[0]


_vec_spec = pl.BlockSpec((1, D), lambda i: (0, 0))


def _tc_s(p, z, degp, b, g, be):
    return pl.pallas_call(
        _s_body,
        grid=(NBLK,),
        in_specs=[_p_spec, _row_spec, _degp_spec, _vec_spec, _vec_spec, _vec_spec],
        out_specs=[_row_spec, _ac_spec],
        out_shape=[
            jax.ShapeDtypeStruct((N, D), F32),
            jax.ShapeDtypeStruct((2, D), F32),
        ],
        scratch_shapes=[pltpu.VMEM((1, D), F32), pltpu.VMEM((1, D), F32)],
    )(p, z, degp, b.reshape(1, D), g.reshape(1, D), be.reshape(1, D))


def _head_body(sums_ref, cnt_ref, maxp_ref, w_ref, b_ref, out_ref):
    sums = sums_ref[0] + sums_ref[1]
    cnt = cnt_ref[0, :] + cnt_ref[1, :]
    mean = sums / jnp.maximum(cnt, 1.0)[:, None]
    xmax = jnp.max(maxp_ref[...], axis=0)
    hm = _dot(mean, w_ref[pl.ds(0, D), :])
    hx = _dot(xmax, w_ref[pl.ds(D, D), :])
    logits = hm + hx + b_ref[0, :][None, :]
    mx = jnp.max(logits, axis=1, keepdims=True)
    lse = mx + jnp.log(jnp.sum(jnp.exp(logits - mx), axis=1, keepdims=True))
    out_ref[...] = logits - lse


def _tc_head(sums, cnt, maxp, linW, linb):
    return pl.pallas_call(
        _head_body,
        out_shape=jax.ShapeDtypeStruct((G, 2), F32),
    )(sums, cnt, maxp, linW, linb.reshape(1, 2))


def kernel(x, edge_index, batch, same_date, W0, b0, W1, b1, W2, b2, W3, b3,
           g0, be0, g1, be1, g2, be2, g3, be3, linW, linb):
    npad = NCHP * CH - E
    row2 = jnp.concatenate([edge_index[0], jnp.zeros((npad,), jnp.int32)]).reshape(NCHP, CH)
    col2 = jnp.concatenate([edge_index[1], jnp.full((npad,), DUMMY, jnp.int32)]).reshape(NCHP, CH)
    sd2 = jnp.concatenate([same_date, jnp.ones((npad,), jnp.int32)]).reshape(NCHP, CH)
    degp, gidx2 = _sc_deg(row2, col2, sd2)
    degp = degp[:, :N].reshape(NC, NBLK, BLK).transpose(1, 0, 2)

    layers = [(W0, b0, g0, be0), (W1, b1, g1, be1),
              (W2, b2, g2, be2), (W3, b3, g3, be3)]

    z, u = _tc_p0(x, degp, W0)
    for li in range(4):
        _, b, g, be = layers[li]
        p = _sc_agg(u.reshape(2 * N, D), gidx2, col2)
        if li < 3:
            Wn = layers[li + 1][0]
            z, u, _ = _tc_sp(p, z, degp, b, g, be, Wn)
        else:
            agg, ac = _tc_s(p, z, degp, b, g, be)
    batch_p = jnp.pad(batch, (0, NP - N), constant_values=G)
    sums, cnt, maxp = _sc_pool(agg, ac, batch_p)
    return _tc_head(sums, cnt, maxp, linW, linb)
